# Initial kernel scaffold; baseline (speedup 1.0000x reference)
import jax
import jax.numpy as jnp
from jax.experimental import pallas as pl


def kernel(token_ids, summary_mask, table):
    raise NotImplementedError("write your pallas kernel here")


# speedup vs baseline: 3.3177x; 3.3177x over previous
import functools

import jax
import jax.numpy as jnp
from jax import lax
from jax.experimental import pallas as pl
from jax.experimental.pallas import tpu as pltpu
from jax.experimental.pallas import tpu_sc as plsc

VOCAB = 100000
B = 4096
L = 50
D = 64

NC = 2
NS = 16
NLANE = 16
NW = NC * NS
B_PER_W = B // NW
ROWS_PER_W = B_PER_W * L
CHUNK = 128
NCHUNK = ROWS_PER_W // CHUNK
DUMP = NS * B_PER_W
ACC_ROWS = DUMP + 8


def _body(tok_hbm, msk_hbm, table_hbm, summary_hbm, toks_hbm,
          tokv, mskv, sidx, rows, ones, accv, cntv, acc_sh, cnt_sh, sem):
    c = lax.axis_index("c")
    s = lax.axis_index("s")
    wid = c * NS + s
    row0 = wid * ROWS_PER_W
    lb0 = s * B_PER_W

    zero16 = jnp.zeros((NLANE,), jnp.float32)
    one16 = jnp.ones((NLANE,), jnp.float32)

    def init_b(j, _):
        for d in range(D // NLANE):
            accv[j, pl.ds(d * NLANE, NLANE)] = zero16
        cntv[j, pl.ds(0, NLANE)] = zero16
        ones[j, pl.ds(0, NLANE)] = one16
        return 0
    lax.fori_loop(0, B_PER_W, init_b, 0)

    pltpu.sync_copy(accv, acc_sh.at[pl.ds(lb0, B_PER_W)])
    pltpu.sync_copy(cntv, cnt_sh.at[pl.ds(lb0, B_PER_W)])

    def chunk_body(g, _):
        rbase = row0 + g * CHUNK
        pltpu.sync_copy(tok_hbm.at[pl.ds(rbase, CHUNK)], tokv)
        pltpu.sync_copy(msk_hbm.at[pl.ds(rbase, CHUNK)], mskv)
        pltpu.async_copy(table_hbm.at[tokv], rows, sem).wait()
        pltpu.sync_copy(rows, toks_hbm.at[pl.ds(rbase, CHUNK)])
        lrow0 = g * CHUNK
        dump_v = jnp.full((NLANE,), DUMP, jnp.int32)
        lane = lax.iota(jnp.int32, NLANE)
        mul_v = jnp.full((NLANE,), 5243, jnp.int32)
        shr_v = jnp.full((NLANE,), 18, jnp.int32)
        for i in range(CHUNK // NLANE):
            m = mskv[pl.ds(i * NLANE, NLANE)]
            r = lane + jnp.full((NLANE,), lrow0 + i * NLANE, jnp.int32)
            lb = ((r * mul_v) >> shr_v) + jnp.full((NLANE,), lb0, jnp.int32)
            sidx[pl.ds(i * NLANE, NLANE)] = jnp.where(m != 0, lb, dump_v)
        pltpu.sync_copy(rows, acc_sh.at[sidx], add=True)
        pltpu.sync_copy(ones, cnt_sh.at[sidx], add=True)
        return 0
    lax.fori_loop(0, NCHUNK, chunk_body, 0)

    pltpu.sync_copy(acc_sh.at[pl.ds(lb0, B_PER_W)], accv)
    pltpu.sync_copy(cnt_sh.at[pl.ds(lb0, B_PER_W)], cntv)

    def sum_row(j, _):
        cv = cntv[j, pl.ds(0, NLANE)]
        inv = one16 / jnp.maximum(cv, one16)
        for d in range(D // NLANE):
            accv[j, pl.ds(d * NLANE, NLANE)] = (
                accv[j, pl.ds(d * NLANE, NLANE)] * inv)
        return 0
    lax.fori_loop(0, B_PER_W, sum_row, 0)

    pltpu.sync_copy(accv, summary_hbm.at[pl.ds(wid * B_PER_W, B_PER_W)])


_mesh = plsc.VectorSubcoreMesh(core_axis_name="c", subcore_axis_name="s",
                               num_cores=NC, num_subcores=NS)

_glove = functools.partial(
    pl.kernel,
    compiler_params=pltpu.CompilerParams(use_tc_tiling_on_sc=False),
    out_type=(
        jax.ShapeDtypeStruct((B, D), jnp.float32),
        jax.ShapeDtypeStruct((B * L, D), jnp.float32),
    ),
    mesh=_mesh,
    scratch_types=[
        pltpu.VMEM((CHUNK,), jnp.int32),
        pltpu.VMEM((CHUNK,), jnp.int32),
        pltpu.VMEM((CHUNK,), jnp.int32),
        pltpu.VMEM((CHUNK, D), jnp.float32),
        pltpu.VMEM((B_PER_W, NLANE), jnp.float32),
        pltpu.VMEM((B_PER_W, D), jnp.float32),
        pltpu.VMEM((B_PER_W, NLANE), jnp.float32),
        pltpu.VMEM_SHARED((ACC_ROWS, D), jnp.float32),
        pltpu.VMEM_SHARED((ACC_ROWS, NLANE), jnp.float32),
        pltpu.SemaphoreType.DMA,
    ],
)(_body)


@jax.jit
def kernel(token_ids, summary_mask, table):
    tok_flat = token_ids.reshape(-1).astype(jnp.int32)
    msk_flat = summary_mask.reshape(-1).astype(jnp.int32)
    summary, toks_flat = _glove(tok_flat, msk_flat, table)
    return summary, toks_flat.reshape(B, L, D)
